# baseline (speedup 1.0000x reference)
import functools

import numpy as np
import jax
import jax.numpy as jnp
from jax import lax
from jax.experimental import pallas as pl
from jax.experimental.pallas import tpu as pltpu
from jax.experimental.pallas import tpu_sc as plsc

B = 1024
N = 128
K = N * (N + 1) // 2
LANES = 16
G = K // LANES

NUM_CORES = 2
NUM_SUBCORES = 16
NW = NUM_CORES * NUM_SUBCORES
B_PER_W = B // NW

_iu0, _iu1 = np.triu_indices(N)
_FLAT_IDX = (_iu0.astype(np.int32) * N + _iu1.astype(np.int32))


def _sc_body(in_hbm, idx_hbm, out_hbm, idx_v, stage_v, out_v):
    wid = lax.axis_index("s") * NUM_CORES + lax.axis_index("c")
    pltpu.sync_copy(idx_hbm, idx_v)

    def per_matrix(bi, carry):
        b = wid * B_PER_W + bi
        pltpu.sync_copy(in_hbm.at[b], stage_v)

        def per_group(g, c):
            base = g * LANES
            idx = idx_v[pl.ds(base, LANES)]
            out_v[pl.ds(base, LANES)] = plsc.load_gather(stage_v, [idx])
            return c

        lax.fori_loop(0, G, per_group, 0)
        pltpu.sync_copy(out_v, out_hbm.at[b])
        return carry

    lax.fori_loop(0, B_PER_W, per_matrix, 0)


def kernel(input):
    x = input.reshape(B, N * N)
    idx = jnp.asarray(_FLAT_IDX)
    mesh = plsc.VectorSubcoreMesh(core_axis_name="c", subcore_axis_name="s")
    f = pl.kernel(
        _sc_body,
        mesh=mesh,
        out_type=jax.ShapeDtypeStruct((B, K), jnp.float32),
        scratch_types=[
            pltpu.VMEM((K,), jnp.int32),
            pltpu.VMEM((N * N,), jnp.float32),
            pltpu.VMEM((K,), jnp.float32),
        ],
        compiler_params=pltpu.CompilerParams(needs_layout_passes=False),
    )
    return f(x, idx)


# speedup vs baseline: 1.5216x; 1.5216x over previous
import numpy as np
import jax
import jax.numpy as jnp
from jax import lax
from jax.experimental import pallas as pl
from jax.experimental.pallas import tpu as pltpu
from jax.experimental.pallas import tpu_sc as plsc

B = 1024
N = 128
K = N * (N + 1) // 2
LANES = 16

NUM_CORES = 2
NUM_SUBCORES = 16
NW = NUM_CORES * NUM_SUBCORES
B_PER_W = B // NW

_COPIES = []
_o = 0
for _i in range(N):
    _L = N - _i
    _src = _i * N + _i
    for _k in range(0, _L, LANES):
        if _o + _k + LANES <= K:
            _COPIES.append((_src + _k, _o + _k))
    _o += _L

_iu0, _iu1 = np.triu_indices(N)
_FLAT_IDX = (_iu0.astype(np.int32) * N + _iu1.astype(np.int32))
_TAIL_IDX = _FLAT_IDX[K - LANES:]


def _sc_body(in_hbm, tail_hbm, out_hbm, stage_v, out_v, tail_v):
    wid = lax.axis_index("s") * NUM_CORES + lax.axis_index("c")
    pltpu.sync_copy(tail_hbm, tail_v)

    def per_matrix(bi, carry):
        b = wid * B_PER_W + bi
        pltpu.sync_copy(in_hbm.at[b], stage_v)
        for s, d in _COPIES:
            out_v[pl.ds(d, LANES)] = stage_v[pl.ds(s, LANES)]
        out_v[pl.ds(K - LANES, LANES)] = plsc.load_gather(stage_v, [tail_v[...]])
        pltpu.sync_copy(out_v, out_hbm.at[b])
        return carry

    lax.fori_loop(0, B_PER_W, per_matrix, 0)


def kernel(input):
    x = input.reshape(B, N * N)
    tail = jnp.asarray(_TAIL_IDX)
    mesh = plsc.VectorSubcoreMesh(core_axis_name="c", subcore_axis_name="s")
    f = pl.kernel(
        _sc_body,
        mesh=mesh,
        out_type=jax.ShapeDtypeStruct((B, K), jnp.float32),
        scratch_types=[
            pltpu.VMEM((N * N,), jnp.float32),
            pltpu.VMEM((K,), jnp.float32),
            pltpu.VMEM((LANES,), jnp.int32),
        ],
        compiler_params=pltpu.CompilerParams(needs_layout_passes=False),
    )
    return f(x, tail)


# speedup vs baseline: 1.8376x; 1.2077x over previous
import numpy as np
import jax
import jax.numpy as jnp
from jax import lax
from jax.experimental import pallas as pl
from jax.experimental.pallas import tpu as pltpu
from jax.experimental.pallas import tpu_sc as plsc

B = 1024
N = 128
K = N * (N + 1) // 2
LANES = 16

NUM_CORES = 2
NUM_SUBCORES = 16
NW = NUM_CORES * NUM_SUBCORES
B_PER_W = B // NW
HALF_ITERS = B_PER_W // 2

_COPIES = []
_o = 0
for _i in range(N):
    _L = N - _i
    _src = _i * N + _i
    for _k in range(0, _L, LANES):
        if _o + _k + LANES <= K:
            _COPIES.append((_src + _k, _o + _k))
    _o += _L

_iu0, _iu1 = np.triu_indices(N)
_FLAT_IDX = (_iu0.astype(np.int32) * N + _iu1.astype(np.int32))
_TAIL_IDX = _FLAT_IDX[K - LANES:]


def _sc_body(in_hbm, tail_hbm, out_hbm,
             stage0, stage1, outb0, outb1, tail_v,
             si0, si1, so0, so1):
    wid = lax.axis_index("s") * NUM_CORES + lax.axis_index("c")
    base = wid * B_PER_W
    pltpu.sync_copy(tail_hbm, tail_v)

    stages = (stage0, stage1)
    outbs = (outb0, outb1)
    sis = (si0, si1)
    sos = (so0, so1)

    pltpu.make_async_copy(in_hbm.at[base], stage0, si0).start()

    def compact(stage, outb):
        for s, d in _COPIES:
            outb[pl.ds(d, LANES)] = stage[pl.ds(s, LANES)]
        outb[pl.ds(K - LANES, LANES)] = plsc.load_gather(stage, [tail_v[...]])

    def half_iter(it, carry):
        for p in (0, 1):
            t = 2 * it + p
            nxt = t + 1

            if p == 0:
                pltpu.make_async_copy(
                    in_hbm.at[base + nxt], stages[1], si1).start()
            else:
                @pl.when(it < HALF_ITERS - 1)
                def _():
                    pltpu.make_async_copy(
                        in_hbm.at[base + nxt], stages[0], si0).start()

            pltpu.make_async_copy(in_hbm.at[base + t], stages[p], sis[p]).wait()

            @pl.when(it >= 1)
            def _():
                pltpu.make_async_copy(
                    outbs[p], out_hbm.at[base + t - 2], sos[p]).wait()

            compact(stages[p], outbs[p])
            pltpu.make_async_copy(outbs[p], out_hbm.at[base + t], sos[p]).start()
        return carry

    lax.fori_loop(0, HALF_ITERS, half_iter, 0)

    pltpu.make_async_copy(outb0, out_hbm.at[base + B_PER_W - 2], so0).wait()
    pltpu.make_async_copy(outb1, out_hbm.at[base + B_PER_W - 1], so1).wait()


def kernel(input):
    x = input.reshape(B, N * N)
    tail = jnp.asarray(_TAIL_IDX)
    mesh = plsc.VectorSubcoreMesh(core_axis_name="c", subcore_axis_name="s")
    f = pl.kernel(
        _sc_body,
        mesh=mesh,
        out_type=jax.ShapeDtypeStruct((B, K), jnp.float32),
        scratch_types=[
            pltpu.VMEM((N * N,), jnp.float32),
            pltpu.VMEM((N * N,), jnp.float32),
            pltpu.VMEM((K,), jnp.float32),
            pltpu.VMEM((K,), jnp.float32),
            pltpu.VMEM((LANES,), jnp.int32),
            pltpu.SemaphoreType.DMA,
            pltpu.SemaphoreType.DMA,
            pltpu.SemaphoreType.DMA,
            pltpu.SemaphoreType.DMA,
        ],
        compiler_params=pltpu.CompilerParams(needs_layout_passes=False),
    )
    return f(x, tail)


# speedup vs baseline: 2.8843x; 1.5696x over previous
import jax
import jax.numpy as jnp
from jax import lax
from jax.experimental import pallas as pl
from jax.experimental.pallas import tpu as pltpu
from jax.experimental.pallas import tpu_sc as plsc

B = 1024
N = 128
K = N * (N + 1) // 2
LANES = 16

NUM_CORES = 2
NUM_SUBCORES = 16
NW = NUM_CORES * NUM_SUBCORES
B_PER_W = B // NW
HALF_ITERS = B_PER_W // 2

_CHUNKS = []
_starts = [0]
for _i in range(N):
    _starts.append(_starts[-1] + (N - _i))
for _i in range(N - 1, -1, -1):
    _L = N - _i
    _nch = -(-_L // LANES)
    for _m in range(_nch):
        _c = N - LANES * (_m + 1)
        _d = _starts[_i] + _L - LANES * (_m + 1)
        _CHUNKS.append((_i, _c, _d))


def _sc_body(in_hbm, out_hbm, stage0, stage1, outb0, outb1, si0, si1, so0, so1):
    wid = lax.axis_index("s") * NUM_CORES + lax.axis_index("c")
    base = wid * B_PER_W

    stages = (stage0, stage1)
    outbs = (outb0, outb1)
    sis = (si0, si1)
    sos = (so0, so1)

    pltpu.make_async_copy(in_hbm.at[base], stage0, si0).start()

    def compact(stage, outb):
        for i, c, d in _CHUNKS:
            outb[pl.ds(d, LANES)] = stage[i, pl.ds(c, LANES)]

    def half_iter(it, carry):
        for p in (0, 1):
            t = 2 * it + p
            nxt = t + 1

            if p == 0:
                pltpu.make_async_copy(
                    in_hbm.at[base + nxt], stages[1], si1).start()
            else:
                @pl.when(it < HALF_ITERS - 1)
                def _():
                    pltpu.make_async_copy(
                        in_hbm.at[base + nxt], stages[0], si0).start()

            pltpu.make_async_copy(in_hbm.at[base + t], stages[p], sis[p]).wait()

            @pl.when(it >= 1)
            def _():
                pltpu.make_async_copy(
                    outbs[p], out_hbm.at[base + t - 2], sos[p]).wait()

            compact(stages[p], outbs[p])
            pltpu.make_async_copy(outbs[p], out_hbm.at[base + t], sos[p]).start()
        return carry

    lax.fori_loop(0, HALF_ITERS, half_iter, 0)

    pltpu.make_async_copy(outb0, out_hbm.at[base + B_PER_W - 2], so0).wait()
    pltpu.make_async_copy(outb1, out_hbm.at[base + B_PER_W - 1], so1).wait()


def kernel(input):
    mesh = plsc.VectorSubcoreMesh(core_axis_name="c", subcore_axis_name="s")
    f = pl.kernel(
        _sc_body,
        mesh=mesh,
        out_type=jax.ShapeDtypeStruct((B, K), jnp.float32),
        scratch_types=[
            pltpu.VMEM((N, N), jnp.float32),
            pltpu.VMEM((N, N), jnp.float32),
            pltpu.VMEM((K,), jnp.float32),
            pltpu.VMEM((K,), jnp.float32),
            pltpu.SemaphoreType.DMA,
            pltpu.SemaphoreType.DMA,
            pltpu.SemaphoreType.DMA,
            pltpu.SemaphoreType.DMA,
        ],
        compiler_params=pltpu.CompilerParams(needs_layout_passes=False),
    )
    return f(input)


# speedup vs baseline: 2.8899x; 1.0019x over previous
import jax
import jax.numpy as jnp
from jax import lax
from jax.experimental import pallas as pl
from jax.experimental.pallas import tpu as pltpu
from jax.experimental.pallas import tpu_sc as plsc

B = 1024
N = 128
K = N * (N + 1) // 2
LANES = 16

NUM_CORES = 2
NUM_SUBCORES = 16
NW = NUM_CORES * NUM_SUBCORES
B_PER_W = B // NW
HALF_ITERS = B_PER_W // 2

_CHUNKS = []
_starts = [0]
for _i in range(N):
    _starts.append(_starts[-1] + (N - _i))
for _i in range(N - 1, -1, -1):
    _L = N - _i
    _nch = -(-_L // LANES)
    for _m in range(_nch):
        _c = N - LANES * (_m + 1)
        _d = _starts[_i] + _L - LANES * (_m + 1)
        _CHUNKS.append((_i, _c, _d))


def _sc_body(in_hbm, out_hbm, stage0, stage1, outb0, outb1, si0, si1, so0, so1):
    wid = lax.axis_index("s") * NUM_CORES + lax.axis_index("c")
    base = wid * B_PER_W

    stages = (stage0, stage1)
    outbs = (outb0, outb1)
    sis = (si0, si1)
    sos = (so0, so1)

    pltpu.make_async_copy(in_hbm.at[base], stage0, si0).start()

    def compact(stage, outb):
        for i, c, d in _CHUNKS:
            outb[pl.ds(d, LANES)] = stage[i, pl.ds(c, LANES)]

    def half_iter(it, carry):
        for p in (0, 1):
            t = 2 * it + p
            nxt = t + 1

            if p == 0:
                pltpu.make_async_copy(
                    in_hbm.at[base + nxt], stages[1], si1).start()
            else:
                @pl.when(it < HALF_ITERS - 1)
                def _():
                    pltpu.make_async_copy(
                        in_hbm.at[base + nxt], stages[0], si0).start()

            pltpu.make_async_copy(in_hbm.at[base + t], stages[p], sis[p]).wait()

            @pl.when(it >= 1)
            def _():
                pltpu.make_async_copy(
                    outbs[p], out_hbm.at[base + t - 2], sos[p]).wait()

            compact(stages[p], outbs[p])
            pltpu.make_async_copy(outbs[p], out_hbm.at[base + t], sos[p]).start()
        return carry

    lax.fori_loop(0, HALF_ITERS, half_iter, 0)

    pltpu.make_async_copy(outb0, out_hbm.at[base + B_PER_W - 2], so0).wait()
    pltpu.make_async_copy(outb1, out_hbm.at[base + B_PER_W - 1], so1).wait()


def kernel(input):
    mesh = plsc.VectorSubcoreMesh(core_axis_name="c", subcore_axis_name="s")
    f = pl.kernel(
        _sc_body,
        mesh=mesh,
        out_type=jax.ShapeDtypeStruct((B, K), jnp.float32),
        scratch_types=[
            pltpu.VMEM((N, N), jnp.float32),
            pltpu.VMEM((N, N), jnp.float32),
            pltpu.VMEM((K,), jnp.float32),
            pltpu.VMEM((K,), jnp.float32),
            pltpu.SemaphoreType.DMA,
            pltpu.SemaphoreType.DMA,
            pltpu.SemaphoreType.DMA,
            pltpu.SemaphoreType.DMA,
        ],
        compiler_params=pltpu.CompilerParams(
            needs_layout_passes=False, use_tc_tiling_on_sc=True),
    )
    return f(input)
